# baseline (speedup 1.0000x reference)
import jax
import jax.numpy as jnp
from jax import lax
from jax.experimental import pallas as pl
from jax.experimental.pallas import tpu as pltpu
from jax.experimental.pallas import tpu_sc as plsc

_VOCAB = 1000000
_BATCH = 16384
_SEQ = 50
_DIM = 32
_CLS = 2

_NW = 32
_BPW = _BATCH // _NW
_GRP = 16
_NGRP = _BPW // _GRP
_ROWS = _GRP * _SEQ
_CHUNK = 80
_NCHUNK = _ROWS // _CHUNK

_TC_BLK = 8192
_TC_GRID = -(-_VOCAB // _TC_BLK)


def _fold_kernel(t_ref, wt_ref, o0_ref, o1_ref):
    p = jnp.dot(t_ref[...], wt_ref[...], preferred_element_type=jnp.float32)
    o0_ref[...] = p[:, 0]
    o1_ref[...] = p[:, 1]


def _fold_table(table, wt):
    return pl.pallas_call(
        _fold_kernel,
        grid=(_TC_GRID,),
        in_specs=[
            pl.BlockSpec((_TC_BLK, _DIM), lambda i: (i, 0)),
            pl.BlockSpec((_DIM, _CLS), lambda i: (0, 0)),
        ],
        out_specs=[
            pl.BlockSpec((_TC_BLK,), lambda i: (i,)),
            pl.BlockSpec((_TC_BLK,), lambda i: (i,)),
        ],
        out_shape=[
            jax.ShapeDtypeStruct((_VOCAB,), jnp.float32),
            jax.ShapeDtypeStruct((_VOCAB,), jnp.float32),
        ],
    )(table, wt)


def _qnet_kernel(x_hbm, tw0_hbm, tw1_hbm, bb_hbm, out_hbm,
                 idx_v, r0_a, r1_a, r0_b, r1_b, bb_v, out_v, sem_a, sem_b):
    wid = lax.axis_index("s") * 2 + lax.axis_index("c")

    pltpu.sync_copy(x_hbm.at[wid], idx_v)
    pltpu.sync_copy(bb_hbm, bb_v)

    bb0 = bb_v[0, :]
    bb1 = bb_v[1, :]
    iota = lax.iota(jnp.int32, 16)
    hi2 = iota >= 14
    m15 = iota == 15
    zero = jnp.zeros((16,), jnp.float32)

    def issue(g, r0_v, r1_v, sem):
        for j in range(_NCHUNK):
            row = idx_v.at[g * _NCHUNK + j]
            dst = pl.ds(j * _CHUNK, _CHUNK)
            pltpu.async_copy(tw0_hbm.at[row], r0_v.at[dst], sem)
            pltpu.async_copy(tw1_hbm.at[row], r1_v.at[dst], sem)

    def drain(g, r0_v, r1_v, sem):
        for j in range(_NCHUNK):
            row = idx_v.at[g * _NCHUNK + j]
            dst = pl.ds(j * _CHUNK, _CHUNK)
            pltpu.make_async_copy(tw0_hbm.at[row], r0_v.at[dst], sem).wait()
            pltpu.make_async_copy(tw1_hbm.at[row], r1_v.at[dst], sem).wait()

    def class_sum(r_v, base):
        v = r_v[pl.ds(base, 16)] + r_v[pl.ds(base + 16, 16)]
        v = v + r_v[pl.ds(base + 32, 16)]
        t = r_v[pl.ds(base + 34, 16)]
        v = v + jnp.where(hi2, t, zero)
        return jnp.cumsum(v)

    def compute(g, r0_v, r1_v):
        def batch_body(bi, c2):
            base = bi * _SEQ
            cs0 = class_sum(r0_v, base) + bb0
            cs1 = class_sum(r1_v, base) + bb1
            o = (g * _GRP + bi) * _CLS
            plsc.store_compressed(out_v.at[pl.ds(o, 16)], cs0, mask=m15)
            plsc.store_compressed(out_v.at[pl.ds(o + 1, 16)], cs1, mask=m15)
            return c2

        lax.fori_loop(0, _GRP, batch_body, 0)

    bufs = ((r0_a, r1_a, sem_a), (r0_b, r1_b, sem_b))

    issue(0, r0_a, r1_a, sem_a)

    def pair_body(j, carry):
        for p in (0, 1):
            g = j * 2 + p
            r0_v, r1_v, sem = bufs[p]
            n0, n1, nsem = bufs[1 - p]
            drain(g, r0_v, r1_v, sem)
            issue(g + 1, n0, n1, nsem)
            compute(g, r0_v, r1_v)
        return carry

    lax.fori_loop(0, _NGRP // 2 - 1, pair_body, 0)

    drain(_NGRP - 2, r0_a, r1_a, sem_a)
    issue(_NGRP - 1, r0_b, r1_b, sem_b)
    compute(_NGRP - 2, r0_a, r1_a)
    drain(_NGRP - 1, r0_b, r1_b, sem_b)
    compute(_NGRP - 1, r0_b, r1_b)

    pltpu.sync_copy(out_v.at[pl.ds(0, _BPW * _CLS)],
                    out_hbm.at[pl.ds(wid * _BPW * _CLS, _BPW * _CLS)])


def kernel(x, table, W, b):
    wt = (W / float(_SEQ)).T
    tw0, tw1 = _fold_table(table, wt)
    xr = x.reshape(_NW, _NGRP * _NCHUNK, _CHUNK)
    bb = jnp.broadcast_to(b[:, None], (_CLS, 16))

    mesh = plsc.VectorSubcoreMesh(core_axis_name="c", subcore_axis_name="s")
    f = pl.kernel(
        _qnet_kernel,
        mesh=mesh,
        compiler_params=pltpu.CompilerParams(
            needs_layout_passes=False, use_tc_tiling_on_sc=False),
        out_type=jax.ShapeDtypeStruct((_BATCH * _CLS,), jnp.float32),
        scratch_types=[
            pltpu.VMEM((_NGRP * _NCHUNK, _CHUNK), jnp.int32),
            pltpu.VMEM((_ROWS,), jnp.float32),
            pltpu.VMEM((_ROWS,), jnp.float32),
            pltpu.VMEM((_ROWS,), jnp.float32),
            pltpu.VMEM((_ROWS,), jnp.float32),
            pltpu.VMEM((_CLS, 16), jnp.float32),
            pltpu.VMEM((_BPW * _CLS + 16,), jnp.float32),
            pltpu.SemaphoreType.DMA,
            pltpu.SemaphoreType.DMA,
        ],
    )
    return f(xr, tw0, tw1, bb).reshape(_BATCH, _CLS)


# speedup vs baseline: 1.0483x; 1.0483x over previous
import jax
import jax.numpy as jnp
from jax import lax
from jax.experimental import pallas as pl
from jax.experimental.pallas import tpu as pltpu
from jax.experimental.pallas import tpu_sc as plsc

_VOCAB = 1000000
_BATCH = 16384
_SEQ = 50
_DIM = 32
_CLS = 2

_NW = 32
_BPW = _BATCH // _NW
_GRP = 16
_NGRP = _BPW // _GRP
_ROWS = _GRP * _SEQ
_CHUNK = 80
_NCHUNK = _ROWS // _CHUNK

_TC_PACK = 4
_TC_NROW = _VOCAB // _TC_PACK
_TC_BLK = 4096
_TC_GRID = -(-_TC_NROW // _TC_BLK)


def _fold_kernel(t_ref, w_ref, o0_ref, o1_ref):
    p = jnp.dot(t_ref[...], w_ref[...], preferred_element_type=jnp.float32)
    o0_ref[...] = p[:, :_TC_PACK]
    o1_ref[...] = p[:, _TC_PACK:]


def _fold_table(table4, w48):
    return pl.pallas_call(
        _fold_kernel,
        grid=(_TC_GRID,),
        in_specs=[
            pl.BlockSpec((_TC_BLK, _TC_PACK * _DIM), lambda i: (i, 0)),
            pl.BlockSpec((_TC_PACK * _DIM, 2 * _TC_PACK), lambda i: (0, 0)),
        ],
        out_specs=[
            pl.BlockSpec((_TC_BLK, _TC_PACK), lambda i: (i, 0)),
            pl.BlockSpec((_TC_BLK, _TC_PACK), lambda i: (i, 0)),
        ],
        out_shape=[
            jax.ShapeDtypeStruct((_TC_NROW, _TC_PACK), jnp.float32),
            jax.ShapeDtypeStruct((_TC_NROW, _TC_PACK), jnp.float32),
        ],
    )(table4, w48)


def _qnet_kernel(x_hbm, tw0_hbm, tw1_hbm, bb_hbm, out_hbm,
                 idx_v, r0_a, r1_a, r0_b, r1_b, bb_v, out_v, sem_a, sem_b):
    wid = lax.axis_index("s") * 2 + lax.axis_index("c")

    pltpu.sync_copy(x_hbm.at[wid], idx_v)
    pltpu.sync_copy(bb_hbm, bb_v)

    bb0 = bb_v[0, :]
    bb1 = bb_v[1, :]
    iota = lax.iota(jnp.int32, 16)
    hi2 = iota >= 14
    m15 = iota == 15
    zero = jnp.zeros((16,), jnp.float32)

    def issue(g, r0_v, r1_v, sem):
        for j in range(_NCHUNK):
            row = idx_v.at[g * _NCHUNK + j]
            dst = pl.ds(j * _CHUNK, _CHUNK)
            pltpu.async_copy(tw0_hbm.at[row], r0_v.at[dst], sem)
            pltpu.async_copy(tw1_hbm.at[row], r1_v.at[dst], sem)

    def drain(g, r0_v, r1_v, sem):
        for j in range(_NCHUNK):
            row = idx_v.at[g * _NCHUNK + j]
            dst = pl.ds(j * _CHUNK, _CHUNK)
            pltpu.make_async_copy(tw0_hbm.at[row], r0_v.at[dst], sem).wait()
            pltpu.make_async_copy(tw1_hbm.at[row], r1_v.at[dst], sem).wait()

    def class_sum(r_v, base):
        v = r_v[pl.ds(base, 16)] + r_v[pl.ds(base + 16, 16)]
        v = v + r_v[pl.ds(base + 32, 16)]
        t = r_v[pl.ds(base + 34, 16)]
        v = v + jnp.where(hi2, t, zero)
        return jnp.cumsum(v)

    def compute(g, r0_v, r1_v):
        def batch_body(bi, c2):
            base = bi * _SEQ
            cs0 = class_sum(r0_v, base) + bb0
            cs1 = class_sum(r1_v, base) + bb1
            o = (g * _GRP + bi) * _CLS
            plsc.store_compressed(out_v.at[pl.ds(o, 16)], cs0, mask=m15)
            plsc.store_compressed(out_v.at[pl.ds(o + 1, 16)], cs1, mask=m15)
            return c2

        lax.fori_loop(0, _GRP, batch_body, 0)

    bufs = ((r0_a, r1_a, sem_a), (r0_b, r1_b, sem_b))

    issue(0, r0_a, r1_a, sem_a)

    def pair_body(j, carry):
        for p in (0, 1):
            g = j * 2 + p
            r0_v, r1_v, sem = bufs[p]
            n0, n1, nsem = bufs[1 - p]
            drain(g, r0_v, r1_v, sem)
            issue(g + 1, n0, n1, nsem)
            compute(g, r0_v, r1_v)
        return carry

    lax.fori_loop(0, _NGRP // 2 - 1, pair_body, 0)

    drain(_NGRP - 2, r0_a, r1_a, sem_a)
    issue(_NGRP - 1, r0_b, r1_b, sem_b)
    compute(_NGRP - 2, r0_a, r1_a)
    drain(_NGRP - 1, r0_b, r1_b, sem_b)
    compute(_NGRP - 1, r0_b, r1_b)

    pltpu.sync_copy(out_v.at[pl.ds(0, _BPW * _CLS)],
                    out_hbm.at[pl.ds(wid * _BPW * _CLS, _BPW * _CLS)])


def kernel(x, table, W, b):
    wt = (W / float(_SEQ)).T
    w48 = jnp.einsum("kj,dc->kdcj", jnp.eye(_TC_PACK, dtype=jnp.float32),
                     wt).reshape(_TC_PACK * _DIM, 2 * _TC_PACK)
    table4 = table.reshape(_TC_NROW, _TC_PACK * _DIM)
    t0, t1 = _fold_table(table4, w48)
    tw0 = t0.reshape(_VOCAB)
    tw1 = t1.reshape(_VOCAB)
    xr = x.reshape(_NW, _NGRP * _NCHUNK, _CHUNK)
    bb = jnp.broadcast_to(b[:, None], (_CLS, 16))

    mesh = plsc.VectorSubcoreMesh(core_axis_name="c", subcore_axis_name="s")
    f = pl.kernel(
        _qnet_kernel,
        mesh=mesh,
        compiler_params=pltpu.CompilerParams(
            needs_layout_passes=False, use_tc_tiling_on_sc=False),
        out_type=jax.ShapeDtypeStruct((_BATCH * _CLS,), jnp.float32),
        scratch_types=[
            pltpu.VMEM((_NGRP * _NCHUNK, _CHUNK), jnp.int32),
            pltpu.VMEM((_ROWS,), jnp.float32),
            pltpu.VMEM((_ROWS,), jnp.float32),
            pltpu.VMEM((_ROWS,), jnp.float32),
            pltpu.VMEM((_ROWS,), jnp.float32),
            pltpu.VMEM((_CLS, 16), jnp.float32),
            pltpu.VMEM((_BPW * _CLS + 16,), jnp.float32),
            pltpu.SemaphoreType.DMA,
            pltpu.SemaphoreType.DMA,
        ],
    )
    return f(xr, tw0, tw1, bb).reshape(_BATCH, _CLS)


# speedup vs baseline: 1.4338x; 1.3677x over previous
import jax
import jax.numpy as jnp
from jax import lax
from jax.experimental import pallas as pl
from jax.experimental.pallas import tpu as pltpu
from jax.experimental.pallas import tpu_sc as plsc

_VOCAB = 1000000
_BATCH = 16384
_SEQ = 50
_DIM = 32
_CLS = 2

_NW = 32
_BPW = _BATCH // _NW
_GRP = 16
_NGRP = _BPW // _GRP
_ROWS = _GRP * _SEQ
_CHUNK = 80
_NCHUNK = _ROWS // _CHUNK

_TC_PACK = 4
_TC_NROW = _VOCAB // _TC_PACK
_TC_BLK = 4096
_TC_GRID = -(-_TC_NROW // _TC_BLK)


def _fold_kernel(t_ref, w_ref, o0_ref, o1_ref):
    q = lax.dot_general(w_ref[...], t_ref[...], (((0,), (1,)), ((), ())),
                        preferred_element_type=jnp.float32)
    o0_ref[...] = q[:_TC_PACK, :]
    o1_ref[...] = q[_TC_PACK:, :]


def _fold_table(table4, w48):
    return pl.pallas_call(
        _fold_kernel,
        grid=(_TC_GRID,),
        in_specs=[
            pl.BlockSpec((_TC_BLK, _TC_PACK * _DIM), lambda i: (i, 0)),
            pl.BlockSpec((_TC_PACK * _DIM, 2 * _TC_PACK), lambda i: (0, 0)),
        ],
        out_specs=[
            pl.BlockSpec((_TC_PACK, _TC_BLK), lambda i: (0, i)),
            pl.BlockSpec((_TC_PACK, _TC_BLK), lambda i: (0, i)),
        ],
        out_shape=[
            jax.ShapeDtypeStruct((_TC_PACK, _TC_NROW), jnp.float32),
            jax.ShapeDtypeStruct((_TC_PACK, _TC_NROW), jnp.float32),
        ],
    )(table4, w48)


def _qnet_kernel(x_hbm, tw0_hbm, tw1_hbm, bb_hbm, out_hbm,
                 idx_v, r0_a, r1_a, r0_b, r1_b, bb_v, out_v, sem_a, sem_b):
    wid = lax.axis_index("s") * 2 + lax.axis_index("c")

    pltpu.sync_copy(x_hbm.at[wid], idx_v)
    pltpu.sync_copy(bb_hbm, bb_v)

    def remap_row(r, c2):
        for t in range(_CHUNK // 16):
            u = idx_v[r, pl.ds(t * 16, 16)]
            idx_v[r, pl.ds(t * 16, 16)] = (
                (u & 3) * _TC_NROW + lax.shift_right_logical(u, 2))
        return c2

    lax.fori_loop(0, _NGRP * _NCHUNK, remap_row, 0)

    bb0 = bb_v[0, :]
    bb1 = bb_v[1, :]
    iota = lax.iota(jnp.int32, 16)
    hi2 = iota >= 14
    m15 = iota == 15
    zero = jnp.zeros((16,), jnp.float32)

    def issue(g, r0_v, r1_v, sem):
        for j in range(_NCHUNK):
            row = idx_v.at[g * _NCHUNK + j]
            dst = pl.ds(j * _CHUNK, _CHUNK)
            pltpu.async_copy(tw0_hbm.at[row], r0_v.at[dst], sem)
            pltpu.async_copy(tw1_hbm.at[row], r1_v.at[dst], sem)

    def drain(g, r0_v, r1_v, sem):
        for j in range(_NCHUNK):
            row = idx_v.at[g * _NCHUNK + j]
            dst = pl.ds(j * _CHUNK, _CHUNK)
            pltpu.make_async_copy(tw0_hbm.at[row], r0_v.at[dst], sem).wait()
            pltpu.make_async_copy(tw1_hbm.at[row], r1_v.at[dst], sem).wait()

    def class_sum(r_v, base):
        v = r_v[pl.ds(base, 16)] + r_v[pl.ds(base + 16, 16)]
        v = v + r_v[pl.ds(base + 32, 16)]
        t = r_v[pl.ds(base + 34, 16)]
        v = v + jnp.where(hi2, t, zero)
        return jnp.cumsum(v)

    def compute(g, r0_v, r1_v):
        def batch_body(bi, c2):
            base = bi * _SEQ
            cs0 = class_sum(r0_v, base) + bb0
            cs1 = class_sum(r1_v, base) + bb1
            o = (g * _GRP + bi) * _CLS
            plsc.store_compressed(out_v.at[pl.ds(o, 16)], cs0, mask=m15)
            plsc.store_compressed(out_v.at[pl.ds(o + 1, 16)], cs1, mask=m15)
            return c2

        lax.fori_loop(0, _GRP, batch_body, 0)

    bufs = ((r0_a, r1_a, sem_a), (r0_b, r1_b, sem_b))

    issue(0, r0_a, r1_a, sem_a)

    def pair_body(j, carry):
        for p in (0, 1):
            g = j * 2 + p
            r0_v, r1_v, sem = bufs[p]
            n0, n1, nsem = bufs[1 - p]
            drain(g, r0_v, r1_v, sem)
            issue(g + 1, n0, n1, nsem)
            compute(g, r0_v, r1_v)
        return carry

    lax.fori_loop(0, _NGRP // 2 - 1, pair_body, 0)

    drain(_NGRP - 2, r0_a, r1_a, sem_a)
    issue(_NGRP - 1, r0_b, r1_b, sem_b)
    compute(_NGRP - 2, r0_a, r1_a)
    drain(_NGRP - 1, r0_b, r1_b, sem_b)
    compute(_NGRP - 1, r0_b, r1_b)

    pltpu.sync_copy(out_v.at[pl.ds(0, _BPW * _CLS)],
                    out_hbm.at[pl.ds(wid * _BPW * _CLS, _BPW * _CLS)])


def kernel(x, table, W, b):
    wt = (W / float(_SEQ)).T
    w48 = jnp.einsum("kj,dc->kdcj", jnp.eye(_TC_PACK, dtype=jnp.float32),
                     wt).reshape(_TC_PACK * _DIM, 2 * _TC_PACK)
    table4 = table.reshape(_TC_NROW, _TC_PACK * _DIM)
    t0, t1 = _fold_table(table4, w48)
    tw0 = t0.reshape(_VOCAB)
    tw1 = t1.reshape(_VOCAB)
    xr = x.reshape(_NW, _NGRP * _NCHUNK, _CHUNK)
    bb = jnp.broadcast_to(b[:, None], (_CLS, 16))

    mesh = plsc.VectorSubcoreMesh(core_axis_name="c", subcore_axis_name="s")
    f = pl.kernel(
        _qnet_kernel,
        mesh=mesh,
        compiler_params=pltpu.CompilerParams(
            needs_layout_passes=False, use_tc_tiling_on_sc=False),
        out_type=jax.ShapeDtypeStruct((_BATCH * _CLS,), jnp.float32),
        scratch_types=[
            pltpu.VMEM((_NGRP * _NCHUNK, _CHUNK), jnp.int32),
            pltpu.VMEM((_ROWS,), jnp.float32),
            pltpu.VMEM((_ROWS,), jnp.float32),
            pltpu.VMEM((_ROWS,), jnp.float32),
            pltpu.VMEM((_ROWS,), jnp.float32),
            pltpu.VMEM((_CLS, 16), jnp.float32),
            pltpu.VMEM((_BPW * _CLS + 16,), jnp.float32),
            pltpu.SemaphoreType.DMA,
            pltpu.SemaphoreType.DMA,
        ],
    )
    return f(xr, tw0, tw1, bb).reshape(_BATCH, _CLS)


# speedup vs baseline: 1.5123x; 1.0548x over previous
import jax
import jax.numpy as jnp
from jax import lax
from jax.experimental import pallas as pl
from jax.experimental.pallas import tpu as pltpu
from jax.experimental.pallas import tpu_sc as plsc

_VOCAB = 1000000
_BATCH = 16384
_SEQ = 50
_DIM = 32
_CLS = 2

_NW = 32
_BPW = _BATCH // _NW
_GRP = 16
_NGRP = _BPW // _GRP
_ROWS = _GRP * _SEQ
_CHUNK = 80
_NCHUNK = _ROWS // _CHUNK

_TC_PACK = 4
_TC_NROW = _VOCAB // _TC_PACK
_TC_BLK = 4096
_TC_GRID = -(-_TC_NROW // _TC_BLK)


def _fold_kernel(t_ref, w_ref, o_ref):
    tb = t_ref[...].astype(jnp.bfloat16)
    wb = w_ref[...].astype(jnp.bfloat16)
    q = lax.dot_general(wb, tb, (((0,), (1,)), ((), ())),
                        preferred_element_type=jnp.float32)
    c0 = lax.bitcast_convert_type(
        q[:_TC_PACK, :].astype(jnp.bfloat16), jnp.uint16)
    c1 = lax.bitcast_convert_type(
        q[_TC_PACK:, :].astype(jnp.bfloat16), jnp.uint16)
    w32 = c0.astype(jnp.uint32) | (c1.astype(jnp.uint32) << 16)
    o_ref[...] = lax.bitcast_convert_type(w32, jnp.int32)


def _fold_table(table4, w48):
    return pl.pallas_call(
        _fold_kernel,
        grid=(_TC_GRID,),
        in_specs=[
            pl.BlockSpec((_TC_BLK, _TC_PACK * _DIM), lambda i: (i, 0)),
            pl.BlockSpec((_TC_PACK * _DIM, 2 * _TC_PACK), lambda i: (0, 0)),
        ],
        out_specs=pl.BlockSpec((_TC_PACK, _TC_BLK), lambda i: (0, i)),
        out_shape=jax.ShapeDtypeStruct((_TC_PACK, _TC_NROW), jnp.int32),
    )(table4, w48)


def _qnet_kernel(x_hbm, tw_hbm, bb_hbm, out_hbm,
                 idx_v, r_a, r_b, bb_v, out_v, sem_a, sem_b):
    wid = lax.axis_index("s") * 2 + lax.axis_index("c")

    pltpu.sync_copy(x_hbm.at[wid], idx_v)
    pltpu.sync_copy(bb_hbm, bb_v)

    def remap_row(r, c2):
        for t in range(_CHUNK // 16):
            u = idx_v[r, pl.ds(t * 16, 16)]
            idx_v[r, pl.ds(t * 16, 16)] = (
                (u & 3) * _TC_NROW + lax.shift_right_logical(u, 2))
        return c2

    lax.fori_loop(0, _NGRP * _NCHUNK, remap_row, 0)

    bb0 = bb_v[0, :]
    bb1 = bb_v[1, :]
    iota = lax.iota(jnp.int32, 16)
    hi2 = iota >= 14
    m15 = iota == 15
    zeroi = jnp.zeros((16,), jnp.int32)

    def issue(g, r_v, sem):
        for j in range(_NCHUNK):
            pltpu.async_copy(
                tw_hbm.at[idx_v.at[g * _NCHUNK + j]],
                r_v.at[pl.ds(j * _CHUNK, _CHUNK)],
                sem)

    def drain(g, r_v, sem):
        for j in range(_NCHUNK):
            pltpu.make_async_copy(
                tw_hbm.at[idx_v.at[g * _NCHUNK + j]],
                r_v.at[pl.ds(j * _CHUNK, _CHUNK)],
                sem).wait()

    def unpack2(w):
        return plsc.unpack(plsc.bitcast(w, jnp.bfloat16),
                           format=plsc.PackFormat.INTERLEAVED)

    def compute(g, r_v):
        def batch_body(bi, c2):
            base = bi * _SEQ
            a0, b0 = unpack2(r_v[pl.ds(base, 16)])
            a1, b1 = unpack2(r_v[pl.ds(base + 16, 16)])
            a2, b2 = unpack2(r_v[pl.ds(base + 32, 16)])
            t = jnp.where(hi2, r_v[pl.ds(base + 34, 16)], zeroi)
            a3, b3 = unpack2(t)
            cs0 = jnp.cumsum((a0 + a1) + (a2 + a3)) + bb0
            cs1 = jnp.cumsum((b0 + b1) + (b2 + b3)) + bb1
            o = (g * _GRP + bi) * _CLS
            plsc.store_compressed(out_v.at[pl.ds(o, 16)], cs0, mask=m15)
            plsc.store_compressed(out_v.at[pl.ds(o + 1, 16)], cs1, mask=m15)
            return c2

        lax.fori_loop(0, _GRP, batch_body, 0)

    bufs = ((r_a, sem_a), (r_b, sem_b))

    issue(0, r_a, sem_a)

    def pair_body(j, carry):
        for p in (0, 1):
            g = j * 2 + p
            r_v, sem = bufs[p]
            n_v, nsem = bufs[1 - p]
            drain(g, r_v, sem)
            issue(g + 1, n_v, nsem)
            compute(g, r_v)
        return carry

    lax.fori_loop(0, _NGRP // 2 - 1, pair_body, 0)

    drain(_NGRP - 2, r_a, sem_a)
    issue(_NGRP - 1, r_b, sem_b)
    compute(_NGRP - 2, r_a)
    drain(_NGRP - 1, r_b, sem_b)
    compute(_NGRP - 1, r_b)

    pltpu.sync_copy(out_v.at[pl.ds(0, _BPW * _CLS)],
                    out_hbm.at[pl.ds(wid * _BPW * _CLS, _BPW * _CLS)])


def kernel(x, table, W, b):
    wt = (W / float(_SEQ)).T
    w48 = jnp.einsum("kj,dc->kdcj", jnp.eye(_TC_PACK, dtype=jnp.float32),
                     wt).reshape(_TC_PACK * _DIM, 2 * _TC_PACK)
    table4 = table.reshape(_TC_NROW, _TC_PACK * _DIM)
    twp = _fold_table(table4, w48).reshape(_VOCAB)
    xr = x.reshape(_NW, _NGRP * _NCHUNK, _CHUNK)
    bb = jnp.broadcast_to(b[:, None], (_CLS, 16))

    mesh = plsc.VectorSubcoreMesh(core_axis_name="c", subcore_axis_name="s")
    f = pl.kernel(
        _qnet_kernel,
        mesh=mesh,
        compiler_params=pltpu.CompilerParams(
            needs_layout_passes=False, use_tc_tiling_on_sc=False),
        out_type=jax.ShapeDtypeStruct((_BATCH * _CLS,), jnp.float32),
        scratch_types=[
            pltpu.VMEM((_NGRP * _NCHUNK, _CHUNK), jnp.int32),
            pltpu.VMEM((_ROWS,), jnp.int32),
            pltpu.VMEM((_ROWS,), jnp.int32),
            pltpu.VMEM((_CLS, 16), jnp.float32),
            pltpu.VMEM((_BPW * _CLS + 16,), jnp.float32),
            pltpu.SemaphoreType.DMA,
            pltpu.SemaphoreType.DMA,
        ],
    )
    return f(xr, twp, bb).reshape(_BATCH, _CLS)


# speedup vs baseline: 1.6973x; 1.1223x over previous
import jax
import jax.numpy as jnp
from jax import lax
from jax.experimental import pallas as pl
from jax.experimental.pallas import tpu as pltpu
from jax.experimental.pallas import tpu_sc as plsc

_BATCH = 16384
_SEQ = 50
_DIM = 32
_CLS = 2

_NW = 32
_BPW = _BATCH // _NW
_GRP = 16
_NGRP = _BPW // _GRP
_ROWS = _GRP * _SEQ
_CHUNK = 100
_NCHUNK = _ROWS // _CHUNK


def _qnet_kernel(x_hbm, we_hbm, bb_hbm, table_hbm, out_hbm,
                 idx_v, rows_a, rows_b, we_v, bb_v, out_v, sem_a, sem_b):
    wid = lax.axis_index("s") * 2 + lax.axis_index("c")

    pltpu.sync_copy(x_hbm.at[wid], idx_v)
    pltpu.sync_copy(we_hbm, we_v)
    pltpu.sync_copy(bb_hbm, bb_v)

    w00 = we_v[0, 0:16]
    w01 = we_v[0, 16:32]
    w10 = we_v[1, 0:16]
    w11 = we_v[1, 16:32]
    bb0 = bb_v[0, :]
    bb1 = bb_v[1, :]
    lane15 = lax.iota(jnp.int32, 16) == 15

    def issue(g, rows_v, sem):
        for j in range(_NCHUNK):
            pltpu.async_copy(
                table_hbm.at[idx_v.at[g * _NCHUNK + j]],
                rows_v.at[pl.ds(j * _CHUNK, _CHUNK), :],
                sem)

    def drain(g, rows_v, sem):
        for j in range(_NCHUNK):
            pltpu.make_async_copy(
                table_hbm.at[idx_v.at[g * _NCHUNK + j]],
                rows_v.at[pl.ds(j * _CHUNK, _CHUNK), :],
                sem).wait()

    def compute(g, rows_v):
        def batch_body(bi, c2):
            r0 = bi * _SEQ
            acc0 = rows_v[r0, 0:16]
            acc1 = rows_v[r0, 16:32]
            for s in range(1, _SEQ):
                acc0 = acc0 + rows_v[r0 + s, 0:16]
                acc1 = acc1 + rows_v[r0 + s, 16:32]
            cs0 = jnp.cumsum(acc0 * w00 + acc1 * w01) + bb0
            cs1 = jnp.cumsum(acc0 * w10 + acc1 * w11) + bb1
            o = (g * _GRP + bi) * _CLS
            plsc.store_compressed(out_v.at[pl.ds(o, 16)], cs0, mask=lane15)
            plsc.store_compressed(out_v.at[pl.ds(o + 1, 16)], cs1, mask=lane15)
            return c2

        lax.fori_loop(0, _GRP, batch_body, 0)

    bufs = ((rows_a, sem_a), (rows_b, sem_b))

    issue(0, rows_a, sem_a)

    def pair_body(j, carry):
        g0 = j * 2
        for p in (0, 1):
            g = g0 + p
            rows_v, sem = bufs[p]
            nrows, nsem = bufs[1 - p]
            drain(g, rows_v, sem)
            issue(g + 1, nrows, nsem)
            compute(g, rows_v)
        return carry

    lax.fori_loop(0, _NGRP // 2 - 1, pair_body, 0)

    drain(_NGRP - 2, rows_a, sem_a)
    issue(_NGRP - 1, rows_b, sem_b)
    compute(_NGRP - 2, rows_a)
    drain(_NGRP - 1, rows_b, sem_b)
    compute(_NGRP - 1, rows_b)

    pltpu.sync_copy(out_v.at[pl.ds(0, _BPW * _CLS)],
                    out_hbm.at[pl.ds(wid * _BPW * _CLS, _BPW * _CLS)])


def kernel(x, table, W, b):
    xr = x.reshape(_NW, _NGRP * _NCHUNK, _CHUNK)
    we = W / float(_SEQ)
    bb = jnp.broadcast_to(b[:, None], (_CLS, 16))

    mesh = plsc.VectorSubcoreMesh(core_axis_name="c", subcore_axis_name="s")
    f = pl.kernel(
        _qnet_kernel,
        mesh=mesh,
        compiler_params=pltpu.CompilerParams(
            needs_layout_passes=False, use_tc_tiling_on_sc=False),
        out_type=jax.ShapeDtypeStruct((_BATCH * _CLS,), jnp.float32),
        scratch_types=[
            pltpu.VMEM((_NGRP * _NCHUNK, _CHUNK), jnp.int32),
            pltpu.VMEM((_ROWS, _DIM), jnp.float32),
            pltpu.VMEM((_ROWS, _DIM), jnp.float32),
            pltpu.VMEM((_CLS, _DIM), jnp.float32),
            pltpu.VMEM((_CLS, 16), jnp.float32),
            pltpu.VMEM((_BPW * _CLS + 16,), jnp.float32),
            pltpu.SemaphoreType.DMA,
            pltpu.SemaphoreType.DMA,
        ],
    )
    return f(xr, we, bb, table).reshape(_BATCH, _CLS)
